# baseline (speedup 1.0000x reference)
import jax
import jax.numpy as jnp
from jax import lax
from jax.experimental import pallas as pl
from jax.experimental.pallas import tpu as pltpu
from jax.experimental.pallas import tpu_sc as plsc

BATCH = 4096
SEQ = 200
DIM = 64
NC, NS, L = 2, 16, 16
NW = NC * NS
ROWS_W = BATCH // NW
OUT_S = SEQ - 1
HALF = 2 * OUT_S
IDX_W = ROWS_W * SEQ
NVEC = (HALF + L - 1) // L

_CHUNKS = ((0, 128), (128, 128), (256, 128), (384, HALF - 384))


def _body(ws_hbm, tab_hbm, out_hbm, idx_v, idx2_v, rows_v, sem):
    wid = lax.axis_index("s") * NC + lax.axis_index("c")
    pltpu.sync_copy(ws_hbm.at[pl.ds(wid * IDX_W, IDX_W)], idx_v.at[pl.ds(0, IDX_W)])
    j = lax.iota(jnp.int32, L)
    pat = ((j >> 1) + (j & 1)).reshape(L, 1)
    dnums = lax.GatherDimensionNumbers(
        offset_dims=(), collapsed_slice_dims=(0,), start_index_map=(0,)
    )

    def step(i, carry):
        base = i * SEQ
        for k in range(NVEC):
            window = idx_v[pl.ds(base + 8 * k, L)]
            idx2_v[pl.ds(L * k, L)] = lax.gather(
                window,
                pat,
                dnums,
                slice_sizes=(1,),
                mode=lax.GatherScatterMode.PROMISE_IN_BOUNDS,
            )
        copies = [
            pltpu.async_copy(
                tab_hbm.at[idx2_v.at[pl.ds(off, n)]],
                rows_v.at[pl.ds(off, n)],
                sem,
            )
            for off, n in _CHUNKS
        ]
        for c in copies:
            c.wait()
        pltpu.sync_copy(rows_v, out_hbm.at[wid * ROWS_W + i])
        return carry

    lax.fori_loop(0, ROWS_W, step, 0)


@jax.jit
def kernel(whitespace, embed_table):
    ws_flat = whitespace.reshape(-1).astype(jnp.int32)
    mesh = plsc.VectorSubcoreMesh(
        core_axis_name="c", subcore_axis_name="s", num_cores=NC, num_subcores=NS
    )
    out = pl.kernel(
        _body,
        out_type=jax.ShapeDtypeStruct((BATCH, HALF, DIM), jnp.float32),
        mesh=mesh,
        compiler_params=pltpu.CompilerParams(use_tc_tiling_on_sc=False),
        scratch_types=[
            pltpu.VMEM((IDX_W + 2 * L,), jnp.int32),
            pltpu.VMEM((NVEC * L,), jnp.int32),
            pltpu.VMEM((HALF, DIM), jnp.float32),
            pltpu.SemaphoreType.DMA,
        ],
    )(ws_flat, embed_table)
    return out.reshape(BATCH, OUT_S, 2 * DIM)
